# Initial kernel scaffold; baseline (speedup 1.0000x reference)
import jax
import jax.numpy as jnp
from jax.experimental import pallas as pl


def kernel(ae, pe, se, edge_index, K, params):
    raise NotImplementedError("write your pallas kernel here")


# speedup vs baseline: 8.2264x; 8.2264x over previous
import functools

import jax
import jax.numpy as jnp
import numpy as np
from jax import lax
from jax.experimental import pallas as pl
from jax.experimental.pallas import tpu as pltpu
from jax.experimental.pallas import tpu_sc as plsc

N = 10000
NPAD = 10112
E = 320000
NC, NS = 2, 16
NW = NC * NS
EW = 10112
EPAD = EW * NW
CHUNK = 128
NCHUNK = EW // CHUNK
EWT = EPAD // NS
NCHUNKT = EWT // CHUNK
ROWS_PER_TILE = NPAD // NS
BLK = 632
F = 128
INV_SQRT32 = 1.0 / np.sqrt(32.0)

_i32 = jnp.int32
_f32 = jnp.float32


def _prep_body(ae_ref, pe_ref, se_ref, pe16_ref, se16_ref,
               aew_ref, aeb_ref, pew_ref, peb_ref, sew_ref, seb_ref,
               wq_pe0, wk_pe0, wq_se0, wk_se0, wqg_pe0, wkg_pe0, wqg_se0, wkg_se0,
               wq_pe1, wk_pe1, wq_se1, wk_se1, wqg_pe1, wkg_pe1, wqg_se1, wkg_se1,
               wv0_ref,
               ae0_out, q_out, k_out, g_out, v0_out):
    pe_e = jnp.dot(pe_ref[...], pew_ref[...], preferred_element_type=_f32) + peb_ref[...]
    se_e = jnp.dot(se_ref[...], sew_ref[...], preferred_element_type=_f32) + seb_ref[...]
    pe_a = jnp.dot(pe16_ref[...], pew_ref[...], preferred_element_type=_f32) + peb_ref[...]
    se_a = jnp.dot(se16_ref[...], sew_ref[...], preferred_element_type=_f32) + seb_ref[...]

    ae0 = jnp.dot(ae_ref[...], aew_ref[...], preferred_element_type=_f32) + aeb_ref[...]
    ae0_out[...] = ae0
    v0_out[...] = jnp.dot(ae0, wv0_ref[...], preferred_element_type=_f32)

    wq = [(wq_pe0, wq_se0), (wq_pe1, wq_se1)]
    wk = [(wk_pe0, wk_se0), (wk_pe1, wk_se1)]
    wqg = [(wqg_pe0, wqg_se0), (wqg_pe1, wqg_se1)]
    wkg = [(wkg_pe0, wkg_se0), (wkg_pe1, wkg_se1)]
    for l in range(2):
        q_pe = jnp.dot(pe_e, wq[l][0][...], preferred_element_type=_f32) * INV_SQRT32
        q_se = jnp.dot(se_e, wq[l][1][...], preferred_element_type=_f32) * INV_SQRT32
        k_pe = jnp.dot(pe_e, wk[l][0][...], preferred_element_type=_f32)
        k_se = jnp.dot(se_e, wk[l][1][...], preferred_element_type=_f32)
        q_out[:, 64 * l:64 * l + 32] = q_pe
        q_out[:, 64 * l + 32:64 * l + 64] = q_se
        k_out[:, 64 * l:64 * l + 32] = k_pe
        k_out[:, 64 * l + 32:64 * l + 64] = k_se
        gq_pe = jnp.dot(pe_e, wqg[l][0][...], preferred_element_type=_f32)
        ka_pe = jnp.dot(pe_a, wkg[l][0][...], preferred_element_type=_f32)
        lg_pe = jnp.dot(gq_pe, ka_pe.T, preferred_element_type=_f32) * INV_SQRT32
        g_pe = jax.nn.softmax(lg_pe, axis=-1)
        gq_se = jnp.dot(se_e, wqg[l][1][...], preferred_element_type=_f32)
        ka_se = jnp.dot(se_a, wkg[l][1][...], preferred_element_type=_f32)
        lg_se = jnp.dot(gq_se, ka_se.T, preferred_element_type=_f32) * INV_SQRT32
        g_se = jax.nn.softmax(lg_se, axis=-1)
        g_out[:, 16 * l:16 * l + 16] = 0.5 * (g_pe + g_se)


def _tc_prep(ae_p, pe_p, se_p, pe16, se16, p, wv0):
    full = lambda shp: pl.BlockSpec(shp, lambda i: (0,) * len(shp))
    row = lambda w: pl.BlockSpec((BLK, w), lambda i: (i, 0))
    lay = p['layers']
    ins = [ae_p, pe_p, se_p, pe16, se16,
           p['ae_emb_w'], p['ae_emb_b'].reshape(1, -1),
           p['pe_emb_w'], p['pe_emb_b'].reshape(1, -1),
           p['se_emb_w'], p['se_emb_b'].reshape(1, -1)]
    for l in range(2):
        ins += [lay[l]['Wq_pe'], lay[l]['Wk_pe'], lay[l]['Wq_se'], lay[l]['Wk_se'],
                lay[l]['Wq_gpe'], lay[l]['Wk_gpe'], lay[l]['Wq_gse'], lay[l]['Wk_gse']]
    ins.append(wv0)
    in_specs = [row(F), row(16), row(16),
                full((16, 16)), full((16, 16)),
                full((F, F)), full((1, F)), full((16, 32)), full((1, 32)),
                full((16, 32)), full((1, 32))]
    in_specs += [full((32, 32))] * 16
    in_specs.append(full((F, F)))
    out_shape = [jax.ShapeDtypeStruct((NPAD, F), _f32),
                 jax.ShapeDtypeStruct((NPAD, F), _f32),
                 jax.ShapeDtypeStruct((NPAD, F), _f32),
                 jax.ShapeDtypeStruct((NPAD, 32), _f32),
                 jax.ShapeDtypeStruct((NPAD, F), _f32)]
    out_specs = [row(F), row(F), row(F),
                 row(32), row(F)]
    return pl.pallas_call(
        _prep_body, grid=(NPAD // BLK,), in_specs=in_specs,
        out_specs=out_specs, out_shape=out_shape)(*ins)


def _sc_scores_body(q_hbm, k_hbm, src_hbm, dst_hbm, zt_hbm,
                    t_hbm, den_hbm,
                    idx_s, idx_d, qrows, krows, tbuf, dbuf, stg, den_sh,
                    sem1, sem2):
    c = lax.axis_index("c")
    s = lax.axis_index("s")
    wid = c * NS + s
    ebase = wid * EW
    rbase = s * ROWS_PER_TILE

    pltpu.sync_copy(zt_hbm, stg)
    pltpu.sync_copy(stg, den_sh.at[pl.ds(rbase, ROWS_PER_TILE), :])
    pltpu.sync_copy(zt_hbm.at[pl.ds(0, CHUNK), :], dbuf)
    plsc.subcore_barrier()

    lanes = jnp.arange(16, dtype=_i32)

    def chunk(ci, carry):
        base = ebase + ci * CHUNK
        pltpu.sync_copy(src_hbm.at[pl.ds(base, CHUNK)], idx_s)
        pltpu.sync_copy(dst_hbm.at[pl.ds(base, CHUNK)], idx_d)
        d1 = pltpu.async_copy(q_hbm.at[idx_d], qrows, sem1)
        d2 = pltpu.async_copy(k_hbm.at[idx_s], krows, sem2)
        d1.wait()
        d2.wait()
        for g in range(8):
            rid = lanes + (g * 16)
            acc = [jnp.zeros((16,), _f32) for _ in range(4)]
            for j in range(4):
                for dd in range(32):
                    col = jnp.full((16,), j * 32 + dd, _i32)
                    qv = plsc.load_gather(qrows, [rid, col])
                    kv = plsc.load_gather(krows, [rid, col])
                    acc[j] = acc[j] + qv * kv
            for j in range(4):
                sj = acc[j]
                ej = jnp.exp(jnp.where(sj > 0, sj, 0.2 * sj))
                cj = jnp.full((16,), j, _i32)
                plsc.store_scatter(tbuf, [rid * 4 + j], ej)
                plsc.store_scatter(dbuf, [rid, cj], ej)
        base4 = pl.multiple_of(base * 4, CHUNK * 4)
        pltpu.sync_copy(tbuf, t_hbm.at[pl.ds(base4, CHUNK * 4)])
        pltpu.sync_copy(dbuf, den_sh.at[idx_d], add=True)
        return carry

    lax.fori_loop(0, NCHUNK, chunk, 0)
    plsc.subcore_barrier()
    obase = c * NPAD + rbase
    pltpu.sync_copy(den_sh.at[pl.ds(rbase, ROWS_PER_TILE), :], stg)
    pltpu.sync_copy(stg, den_hbm.at[pl.ds(obase, ROWS_PER_TILE), :])


def _sc_scores(q, k, src_p, dst_p, zt):
    mesh = plsc.VectorSubcoreMesh(core_axis_name="c", subcore_axis_name="s",
                                  num_cores=NC, num_subcores=NS)
    f = pl.kernel(
        _sc_scores_body,
        out_type=[jax.ShapeDtypeStruct((EPAD * 4,), _f32),
                  jax.ShapeDtypeStruct((NC * NPAD, 16), _f32)],
        mesh=mesh,
        compiler_params=pltpu.CompilerParams(needs_layout_passes=False,
                                             use_tc_tiling_on_sc=False),
        scratch_types=[
            pltpu.VMEM((CHUNK,), _i32), pltpu.VMEM((CHUNK,), _i32),
            pltpu.VMEM((CHUNK, F), _f32), pltpu.VMEM((CHUNK, F), _f32),
            pltpu.VMEM((CHUNK * 4,), _f32), pltpu.VMEM((CHUNK, 16), _f32),
            pltpu.VMEM((ROWS_PER_TILE, 16), _f32),
            pltpu.VMEM_SHARED((NPAD, 16), _f32),
            pltpu.SemaphoreType.DMA, pltpu.SemaphoreType.DMA,
        ])
    return f(q, k, src_p, dst_p, zt)


def _r_body(den_ref, r_out):
    d = den_ref[:NPAD, :] + den_ref[NPAD:, :]
    r_out[...] = 0.5 / (d + 1e-16)


def _tc_r(den):
    return pl.pallas_call(
        _r_body,
        in_specs=[pl.BlockSpec((NC * NPAD, 16), lambda: (0, 0))],
        out_specs=pl.BlockSpec((NPAD, 16), lambda: (0, 0)),
        out_shape=jax.ShapeDtypeStruct((NPAD, 16), _f32))(den)


def _sc_agg_body(l, v_hbm, src_hbm, dst_hbm, t_hbm, r_hbm, z_hbm,
                 acc_hbm,
                 idx_s, idx_d, vrows, trows, rrows, wbuf, stg, rstg,
                 acc_sh, r_sh, sem1, sem2):
    c = lax.axis_index("c")
    s = lax.axis_index("s")
    ebase = s * EWT
    rbase = s * ROWS_PER_TILE

    pltpu.sync_copy(z_hbm, stg)
    pltpu.sync_copy(stg, acc_sh.at[pl.ds(rbase, ROWS_PER_TILE), :])
    pltpu.sync_copy(r_hbm.at[pl.ds(rbase, ROWS_PER_TILE), :], rstg)
    pltpu.sync_copy(rstg, r_sh.at[pl.ds(rbase, ROWS_PER_TILE), :])
    plsc.subcore_barrier()

    lanes = jnp.arange(16, dtype=_i32)
    voff = c * NPAD

    def chunk(ci, carry):
        base = ebase + ci * CHUNK
        pltpu.sync_copy(src_hbm.at[pl.ds(base, CHUNK)], idx_s)
        pltpu.sync_copy(dst_hbm.at[pl.ds(base, CHUNK)], idx_d)
        for g in range(8):
            sl = pl.ds(g * 16, 16)
            idx_s[sl] = idx_s[sl] + voff
        d1 = pltpu.async_copy(v_hbm.at[idx_s], vrows, sem1)
        d2 = pltpu.async_copy(r_sh.at[idx_d], rrows, sem2)
        base4 = pl.multiple_of(base * 4, CHUNK * 4)
        pltpu.sync_copy(t_hbm.at[pl.ds(base4, CHUNK * 4)], trows)
        d1.wait()
        d2.wait()
        for g in range(8):
            rid = lanes + (g * 16)
            t_pe = plsc.load_gather(trows, [rid * 4 + 2 * l])
            t_se = plsc.load_gather(trows, [rid * 4 + 2 * l + 1])
            r_pe = plsc.load_gather(rrows, [rid, jnp.full((16,), 2 * l, _i32)])
            r_se = plsc.load_gather(rrows, [rid, jnp.full((16,), 2 * l + 1, _i32)])
            wbuf[pl.ds(g * 16, 16)] = t_pe * r_pe + t_se * r_se

        def scale(e, carry2):
            ecol = jnp.full((16,), e, _i32)
            wv = plsc.load_gather(wbuf, [ecol])
            for kk in range(4):
                cid = lanes + (kk * 16)
                vv = plsc.load_gather(vrows, [ecol, cid])
                plsc.store_scatter(vrows, [ecol, cid], vv * wv)
            return carry2

        lax.fori_loop(0, CHUNK, scale, 0)
        pltpu.sync_copy(vrows, acc_sh.at[idx_d], add=True)
        return carry

    lax.fori_loop(0, NCHUNKT, chunk, 0)
    plsc.subcore_barrier()
    obase = c * NPAD + rbase
    pltpu.sync_copy(acc_sh.at[pl.ds(rbase, ROWS_PER_TILE), :], stg)
    pltpu.sync_copy(stg, acc_hbm.at[pl.ds(obase, ROWS_PER_TILE), :])


def _sc_agg(l, v, src_p, dst_p, t_all, r_tab, z):
    v_split = jnp.concatenate([v[:, :64], v[:, 64:]], axis=0)
    mesh = plsc.VectorSubcoreMesh(core_axis_name="c", subcore_axis_name="s",
                                  num_cores=NC, num_subcores=NS)
    f = pl.kernel(
        functools.partial(_sc_agg_body, l),
        out_type=jax.ShapeDtypeStruct((NC * NPAD, 64), _f32),
        mesh=mesh,
        compiler_params=pltpu.CompilerParams(needs_layout_passes=False,
                                             use_tc_tiling_on_sc=False),
        scratch_types=[
            pltpu.VMEM((CHUNK,), _i32), pltpu.VMEM((CHUNK,), _i32),
            pltpu.VMEM((CHUNK, 64), _f32), pltpu.VMEM((CHUNK * 4,), _f32),
            pltpu.VMEM((CHUNK, 16), _f32), pltpu.VMEM((CHUNK,), _f32),
            pltpu.VMEM((ROWS_PER_TILE, 64), _f32),
            pltpu.VMEM((ROWS_PER_TILE, 16), _f32),
            pltpu.VMEM_SHARED((NPAD, 64), _f32),
            pltpu.VMEM_SHARED((NPAD, 16), _f32),
            pltpu.SemaphoreType.DMA, pltpu.SemaphoreType.DMA,
        ])
    acc = f(v_split, src_p, dst_p, t_all, r_tab, z)
    return jnp.concatenate([acc[:NPAD], acc[NPAD:]], axis=1)


def _int0_body(ae_ref, loc_ref, g_ref, v16_ref, wg_ref, bg_ref, wv1_ref,
               ae1_out, v1_out):
    ae = ae_ref[...]
    local = loc_ref[...]
    gate = jax.nn.sigmoid(jnp.sum(ae * wg_ref[...], axis=-1, keepdims=True)
                          + bg_ref[...])
    glob = jnp.dot(g_ref[...], v16_ref[...], preferred_element_type=_f32)
    ae1 = ae + gate * local + (1.0 - gate) * glob
    ae1_out[...] = ae1
    v1_out[...] = jnp.dot(ae1, wv1_ref[...], preferred_element_type=_f32)


def _int1_body(ae_ref, loc_ref, g_ref, v16_ref, wg_ref, bg_ref,
               w1_ref, b1_ref, w2_ref, b2_ref, w3_ref, b3_ref, out_ref):
    ae = ae_ref[...]
    local = loc_ref[...]
    gate = jax.nn.sigmoid(jnp.sum(ae * wg_ref[...], axis=-1, keepdims=True)
                          + bg_ref[...])
    glob = jnp.dot(g_ref[...], v16_ref[...], preferred_element_type=_f32)
    ae2 = ae + gate * local + (1.0 - gate) * glob
    x = jax.nn.relu(jnp.dot(ae2, w1_ref[...], preferred_element_type=_f32) + b1_ref[...])
    x = jax.nn.relu(jnp.dot(x, w2_ref[...], preferred_element_type=_f32) + b2_ref[...])
    out_ref[...] = jnp.dot(x, w3_ref[...], preferred_element_type=_f32) + b3_ref[...]


def _tc_int0(ae_cur, acc, g_tab, v16, lp, wv1):
    full = lambda shp: pl.BlockSpec(shp, lambda i: (0,) * len(shp))
    row = lambda w: pl.BlockSpec((BLK, w), lambda i: (i, 0))
    gspec = pl.BlockSpec((BLK, 16), lambda i: (i, 0))
    ins = [ae_cur, acc, g_tab[:, :16], v16,
           lp['w_gate'].reshape(1, F), lp['b_gate'].reshape(1, 1), wv1]
    in_specs = [row(F), row(F), gspec,
                full((16, F)), full((1, F)), full((1, 1)), full((F, F))]
    return pl.pallas_call(
        _int0_body, grid=(NPAD // BLK,), in_specs=in_specs,
        out_specs=[row(F), row(F)],
        out_shape=[jax.ShapeDtypeStruct((NPAD, F), _f32),
                   jax.ShapeDtypeStruct((NPAD, F), _f32)])(*ins)


def _tc_int1(ae_cur, acc, g_tab, v16, lp, p):
    full = lambda shp: pl.BlockSpec(shp, lambda i: (0,) * len(shp))
    row = lambda w: pl.BlockSpec((BLK, w), lambda i: (i, 0))
    gspec = pl.BlockSpec((BLK, 16), lambda i: (i, 0))
    ins = [ae_cur, acc, g_tab[:, 16:], v16,
           lp['w_gate'].reshape(1, F), lp['b_gate'].reshape(1, 1),
           p['cls1_w'], p['cls1_b'].reshape(1, -1),
           p['cls2_w'], p['cls2_b'].reshape(1, -1),
           p['cls3_w'], p['cls3_b'].reshape(1, -1)]
    in_specs = [row(F), row(F), gspec,
                full((16, F)), full((1, F)), full((1, 1)),
                full((F, 64)), full((1, 64)), full((64, 32)), full((1, 32)),
                full((32, 10)), full((1, 10))]
    return pl.pallas_call(
        _int1_body, grid=(NPAD // BLK,), in_specs=in_specs,
        out_specs=row(10),
        out_shape=jax.ShapeDtypeStruct((NPAD, 10), _f32))(*ins)


def kernel(ae, pe, se, edge_index, K, params):
    del K
    ae = ae.astype(_f32)
    pe = pe.astype(_f32)
    se = se.astype(_f32)
    pe16 = pe[:16]
    se16 = se[:16]
    ae_p = jnp.pad(ae, ((0, NPAD - N), (0, 0)))
    pe_p = jnp.pad(pe, ((0, NPAD - N), (0, 0)))
    se_p = jnp.pad(se, ((0, NPAD - N), (0, 0)))

    src = edge_index[0].astype(_i32)
    dst = edge_index[1].astype(_i32)
    pad_idx = N + (jnp.arange(EPAD - E, dtype=_i32) % (NPAD - N))
    src_p = jnp.concatenate([src, pad_idx])
    dst_p = jnp.concatenate([dst, pad_idx])

    zt = jnp.zeros((ROWS_PER_TILE, 16), _f32)
    zf = jnp.zeros((ROWS_PER_TILE, 64), _f32)

    lay = params['layers']
    ae0, q_tab, k_tab, g_tab, v0 = _tc_prep(ae_p, pe_p, se_p, pe16, se16,
                                            params, lay[0]['Wv_ae'])
    t_all, den = _sc_scores(q_tab, k_tab, src_p, dst_p, zt)
    r_tab = _tc_r(den)

    acc0 = _sc_agg(0, v0, src_p, dst_p, t_all, r_tab, zf)
    ae1, v1 = _tc_int0(ae0, acc0, g_tab, v0[:16], lay[0], lay[1]['Wv_ae'])

    acc1 = _sc_agg(1, v1, src_p, dst_p, t_all, r_tab, zf)
    out = _tc_int1(ae1, acc1, g_tab, v1[:16], lay[1], params)
    return out[:N]


# speedup vs baseline: 36.6110x; 4.4504x over previous
import functools

import jax
import jax.numpy as jnp
import numpy as np
from jax import lax
from jax.experimental import pallas as pl
from jax.experimental.pallas import tpu as pltpu
from jax.experimental.pallas import tpu_sc as plsc

N = 10000
NPAD = 10112
E = 320000
NC, NS = 2, 16
NW = NC * NS
EW = 10112
EPAD = EW * NW
CHUNK = 128
NCHUNK = EW // CHUNK
EWT = EPAD // NS
NCHUNKT = EWT // CHUNK
ROWS_PER_TILE = NPAD // NS
BLK = 632
F = 128
INV_SQRT32 = 1.0 / np.sqrt(32.0)

_i32 = jnp.int32
_f32 = jnp.float32


def _prep_body(ae_ref, pe_ref, se_ref, pe16_ref, se16_ref,
               aew_ref, aeb_ref, pew_ref, peb_ref, sew_ref, seb_ref,
               wq_pe0, wk_pe0, wq_se0, wk_se0, wqg_pe0, wkg_pe0, wqg_se0, wkg_se0,
               wq_pe1, wk_pe1, wq_se1, wk_se1, wqg_pe1, wkg_pe1, wqg_se1, wkg_se1,
               wv0_ref,
               ae0_out, q_out, k_out, g_out, v0_out):
    pe_e = jnp.dot(pe_ref[...], pew_ref[...], preferred_element_type=_f32) + peb_ref[...]
    se_e = jnp.dot(se_ref[...], sew_ref[...], preferred_element_type=_f32) + seb_ref[...]
    pe_a = jnp.dot(pe16_ref[...], pew_ref[...], preferred_element_type=_f32) + peb_ref[...]
    se_a = jnp.dot(se16_ref[...], sew_ref[...], preferred_element_type=_f32) + seb_ref[...]

    ae0 = jnp.dot(ae_ref[...], aew_ref[...], preferred_element_type=_f32) + aeb_ref[...]
    ae0_out[...] = ae0
    v0_out[...] = jnp.dot(ae0, wv0_ref[...], preferred_element_type=_f32)

    wq = [(wq_pe0, wq_se0), (wq_pe1, wq_se1)]
    wk = [(wk_pe0, wk_se0), (wk_pe1, wk_se1)]
    wqg = [(wqg_pe0, wqg_se0), (wqg_pe1, wqg_se1)]
    wkg = [(wkg_pe0, wkg_se0), (wkg_pe1, wkg_se1)]
    for l in range(2):
        q_pe = jnp.dot(pe_e, wq[l][0][...], preferred_element_type=_f32) * INV_SQRT32
        q_se = jnp.dot(se_e, wq[l][1][...], preferred_element_type=_f32) * INV_SQRT32
        k_pe = jnp.dot(pe_e, wk[l][0][...], preferred_element_type=_f32)
        k_se = jnp.dot(se_e, wk[l][1][...], preferred_element_type=_f32)
        q_out[:, 64 * l:64 * l + 32] = q_pe
        q_out[:, 64 * l + 32:64 * l + 64] = q_se
        k_out[:, 64 * l:64 * l + 32] = k_pe
        k_out[:, 64 * l + 32:64 * l + 64] = k_se
        gq_pe = jnp.dot(pe_e, wqg[l][0][...], preferred_element_type=_f32)
        ka_pe = jnp.dot(pe_a, wkg[l][0][...], preferred_element_type=_f32)
        lg_pe = jnp.dot(gq_pe, ka_pe.T, preferred_element_type=_f32) * INV_SQRT32
        g_pe = jax.nn.softmax(lg_pe, axis=-1)
        gq_se = jnp.dot(se_e, wqg[l][1][...], preferred_element_type=_f32)
        ka_se = jnp.dot(se_a, wkg[l][1][...], preferred_element_type=_f32)
        lg_se = jnp.dot(gq_se, ka_se.T, preferred_element_type=_f32) * INV_SQRT32
        g_se = jax.nn.softmax(lg_se, axis=-1)
        g_out[:, 16 * l:16 * l + 16] = 0.5 * (g_pe + g_se)


def _tc_prep(ae_p, pe_p, se_p, pe16, se16, p, wv0):
    full = lambda shp: pl.BlockSpec(shp, lambda i: (0,) * len(shp))
    row = lambda w: pl.BlockSpec((BLK, w), lambda i: (i, 0))
    lay = p['layers']
    ins = [ae_p, pe_p, se_p, pe16, se16,
           p['ae_emb_w'], p['ae_emb_b'].reshape(1, -1),
           p['pe_emb_w'], p['pe_emb_b'].reshape(1, -1),
           p['se_emb_w'], p['se_emb_b'].reshape(1, -1)]
    for l in range(2):
        ins += [lay[l]['Wq_pe'], lay[l]['Wk_pe'], lay[l]['Wq_se'], lay[l]['Wk_se'],
                lay[l]['Wq_gpe'], lay[l]['Wk_gpe'], lay[l]['Wq_gse'], lay[l]['Wk_gse']]
    ins.append(wv0)
    in_specs = [row(F), row(16), row(16),
                full((16, 16)), full((16, 16)),
                full((F, F)), full((1, F)), full((16, 32)), full((1, 32)),
                full((16, 32)), full((1, 32))]
    in_specs += [full((32, 32))] * 16
    in_specs.append(full((F, F)))
    out_shape = [jax.ShapeDtypeStruct((NPAD, F), _f32),
                 jax.ShapeDtypeStruct((NPAD, F), _f32),
                 jax.ShapeDtypeStruct((NPAD, F), _f32),
                 jax.ShapeDtypeStruct((NPAD, 32), _f32),
                 jax.ShapeDtypeStruct((NPAD, F), _f32)]
    out_specs = [row(F), row(F), row(F),
                 row(32), row(F)]
    return pl.pallas_call(
        _prep_body, grid=(NPAD // BLK,), in_specs=in_specs,
        out_specs=out_specs, out_shape=out_shape)(*ins)


def _sc_scores_body(q_hbm, k_hbm, src_hbm, dst_hbm, zt_hbm,
                    t_hbm, den_hbm,
                    isrc, idst, qr0, qr1, kr0, kr1, tb0, tb1, db0, db1, stg,
                    den_sh,
                    sq0, sq1, sk0, sk1, st0, st1, sd0, sd1):
    c = lax.axis_index("c")
    s = lax.axis_index("s")
    wid = c * NS + s
    ebase = wid * EW
    ibase = wid * NCHUNK
    rbase = s * ROWS_PER_TILE
    qr = [qr0, qr1]
    kr = [kr0, kr1]
    tb = [tb0, tb1]
    db = [db0, db1]
    sq = [sq0, sq1]
    sk = [sk0, sk1]
    st = [st0, st1]
    sd = [sd0, sd1]

    pltpu.sync_copy(zt_hbm, stg)
    pltpu.sync_copy(stg, den_sh.at[pl.ds(rbase, ROWS_PER_TILE), :])
    pltpu.sync_copy(zt_hbm.at[pl.ds(0, CHUNK), :], db0)
    pltpu.sync_copy(zt_hbm.at[pl.ds(0, CHUNK), :], db1)
    pltpu.sync_copy(src_hbm.at[pl.ds(ibase, NCHUNK), :], isrc)
    pltpu.sync_copy(dst_hbm.at[pl.ds(ibase, NCHUNK), :], idst)
    plsc.subcore_barrier()

    lanes = jnp.arange(16, dtype=_i32)

    def issue(n, b):
        pltpu.async_copy(q_hbm.at[idst.at[n]], qr[b], sq[b])
        pltpu.async_copy(k_hbm.at[isrc.at[n]], kr[b], sk[b])

    def wait_rows(b):
        pltpu.make_async_copy(q_hbm.at[pl.ds(0, CHUNK), :], qr[b], sq[b]).wait()
        pltpu.make_async_copy(k_hbm.at[pl.ds(0, CHUNK), :], kr[b], sk[b]).wait()

    def wait_out(b):
        pltpu.make_async_copy(tb[b], t_hbm.at[0], st[b]).wait()
        pltpu.make_async_copy(db[b], den_sh.at[pl.ds(0, CHUNK), :], sd[b]).wait()

    def compute(n, b, guard):
        if guard:
            @pl.when(n >= 2)
            def _():
                wait_out(b)
        else:
            wait_out(b)
        for g in range(8):
            rid = lanes + (g * 16)
            z16 = jnp.zeros((16,), _f32)

            def dotstep(d2, accs):
                a0, a1, a2, a3 = accs
                for u in range(4):
                    d = d2 * 4 + u
                    skew = (lanes + d) & 31
                    aa = [a0, a1, a2, a3]
                    for j in range(4):
                        col = skew + (j * 32)
                        qv = plsc.load_gather(qr[b], [rid, col])
                        kv = plsc.load_gather(kr[b], [rid, col])
                        aa[j] = aa[j] + qv * kv
                    a0, a1, a2, a3 = aa
                return (a0, a1, a2, a3)

            accs = lax.fori_loop(0, 8, dotstep, (z16, z16, z16, z16))
            for j in range(4):
                sj = accs[j]
                ej = jnp.exp(jnp.where(sj > 0, sj, 0.2 * sj))
                cj = jnp.full((16,), j, _i32)
                plsc.store_scatter(tb[b], [rid * 4 + j], ej)
                plsc.store_scatter(db[b], [rid, cj], ej)
        pltpu.async_copy(tb[b], t_hbm.at[ibase + n], st[b])
        pltpu.async_copy(db[b], den_sh.at[idst.at[n]], sd[b], add=True)

    issue(0, 0)

    def pairloop(p, carry):
        n0 = 2 * p
        issue(n0 + 1, 1)
        wait_rows(0)
        compute(n0, 0, guard=True)
        issue(n0 + 2, 0)
        wait_rows(1)
        compute(n0 + 1, 1, guard=True)
        return carry

    lax.fori_loop(0, (NCHUNK - 1) // 2, pairloop, 0)
    wait_rows(0)
    compute(NCHUNK - 1, 0, guard=False)
    wait_out(0)
    wait_out(1)
    plsc.subcore_barrier()
    obase = c * NPAD + rbase
    pltpu.sync_copy(den_sh.at[pl.ds(rbase, ROWS_PER_TILE), :], stg)
    pltpu.sync_copy(stg, den_hbm.at[pl.ds(obase, ROWS_PER_TILE), :])


def _sc_scores(q, k, src2d, dst2d, zt):
    mesh = plsc.VectorSubcoreMesh(core_axis_name="c", subcore_axis_name="s",
                                  num_cores=NC, num_subcores=NS)
    f = pl.kernel(
        _sc_scores_body,
        out_type=[jax.ShapeDtypeStruct((NW * NCHUNK, CHUNK * 4), _f32),
                  jax.ShapeDtypeStruct((NC * NPAD, 16), _f32)],
        mesh=mesh,
        compiler_params=pltpu.CompilerParams(needs_layout_passes=False,
                                             use_tc_tiling_on_sc=False),
        scratch_types=[
            pltpu.VMEM((NCHUNK, CHUNK), _i32), pltpu.VMEM((NCHUNK, CHUNK), _i32),
            pltpu.VMEM((CHUNK, F), _f32), pltpu.VMEM((CHUNK, F), _f32),
            pltpu.VMEM((CHUNK, F), _f32), pltpu.VMEM((CHUNK, F), _f32),
            pltpu.VMEM((CHUNK * 4,), _f32), pltpu.VMEM((CHUNK * 4,), _f32),
            pltpu.VMEM((CHUNK, 16), _f32), pltpu.VMEM((CHUNK, 16), _f32),
            pltpu.VMEM((ROWS_PER_TILE, 16), _f32),
            pltpu.VMEM_SHARED((NPAD, 16), _f32),
        ] + [pltpu.SemaphoreType.DMA] * 8)
    return f(q, k, src2d, dst2d, zt)


def _r_body(den_ref, r_out):
    d = den_ref[:NPAD, :] + den_ref[NPAD:, :]
    r_out[...] = 0.5 / (d + 1e-16)


def _tc_r(den):
    return pl.pallas_call(
        _r_body,
        in_specs=[pl.BlockSpec((NC * NPAD, 16), lambda: (0, 0))],
        out_specs=pl.BlockSpec((NPAD, 16), lambda: (0, 0)),
        out_shape=jax.ShapeDtypeStruct((NPAD, 16), _f32))(den)


def _sc_agg_body(l, v_hbm, src_hbm, dst_hbm, t_hbm, r_hbm, z_hbm,
                 acc_hbm,
                 isrc, idst, vr0, vr1, tr0, tr1, rr0, rr1, wbuf, stg, rstg,
                 rstg2, acc_sh, r_sh,
                 sv0, sv1, sr0, sr1, st0, st1, sa0, sa1):
    c = lax.axis_index("c")
    s = lax.axis_index("s")
    ebase = s * EWT
    ibase = s * NCHUNKT
    rbase = s * ROWS_PER_TILE
    vr = [vr0, vr1]
    tr = [tr0, tr1]
    rr = [rr0, rr1]
    sv = [sv0, sv1]
    sr = [sr0, sr1]
    st = [st0, st1]
    sa = [sa0, sa1]
    QR = ROWS_PER_TILE // 4

    lanes = jnp.arange(16, dtype=_i32)
    voff = c * NPAD
    pltpu.sync_copy(z_hbm, stg)
    for qq in range(4):
        pltpu.sync_copy(stg, acc_sh.at[pl.ds(rbase + qq * QR, QR), :])
        pltpu.sync_copy(r_hbm.at[pl.ds(rbase + qq * QR, QR), :], rstg)
        pltpu.sync_copy(r_hbm.at[pl.ds(NPAD + rbase + qq * QR, QR), :], rstg2)

        def recip(rrow, carry):
            rv = jnp.full((16,), rrow, _i32)
            d0 = plsc.load_gather(rstg, [rv, lanes])
            d1 = plsc.load_gather(rstg2, [rv, lanes])
            plsc.store_scatter(rstg, [rv, lanes], 0.5 / (d0 + d1 + 1e-16))
            return carry

        lax.fori_loop(0, QR, recip, 0)
        pltpu.sync_copy(rstg, r_sh.at[pl.ds(rbase + qq * QR, QR), :])
    pltpu.sync_copy(src_hbm.at[pl.ds(ibase, NCHUNKT), :], isrc)
    pltpu.sync_copy(dst_hbm.at[pl.ds(ibase, NCHUNKT), :], idst)

    def offs(r, carry):
        rv = jnp.full((16,), r, _i32)
        for g in range(8):
            cid = lanes + (g * 16)
            iv = plsc.load_gather(isrc, [rv, cid])
            plsc.store_scatter(isrc, [rv, cid], iv + voff)
        return carry

    lax.fori_loop(0, NCHUNKT, offs, 0)
    plsc.subcore_barrier()

    def issue(n, b):
        pltpu.async_copy(v_hbm.at[isrc.at[n]], vr[b], sv[b])
        pltpu.async_copy(r_sh.at[idst.at[n]], rr[b], sr[b])
        pltpu.async_copy(t_hbm.at[ibase + n], tr[b], st[b])

    def wait_set(b):
        pltpu.make_async_copy(v_hbm.at[pl.ds(0, CHUNK), :], vr[b], sv[b]).wait()
        pltpu.make_async_copy(r_hbm.at[pl.ds(0, CHUNK), :], rr[b], sr[b]).wait()
        pltpu.make_async_copy(t_hbm.at[0], tr[b], st[b]).wait()

    def wait_acc(b):
        pltpu.make_async_copy(vr[b], acc_sh.at[pl.ds(0, CHUNK), :], sa[b]).wait()

    def compute(n, b, guard):
        if guard:
            @pl.when(n >= 2)
            def _():
                wait_acc(b)
        else:
            wait_acc(b)
        for g in range(8):
            rid = lanes + (g * 16)
            t_pe = plsc.load_gather(tr[b], [rid * 4 + 2 * l])
            t_se = plsc.load_gather(tr[b], [rid * 4 + 2 * l + 1])
            r_pe = plsc.load_gather(rr[b], [rid, jnp.full((16,), 2 * l, _i32)])
            r_se = plsc.load_gather(rr[b], [rid, jnp.full((16,), 2 * l + 1, _i32)])

            wbuf[pl.ds(g * 16, 16)] = t_pe * r_pe + t_se * r_se

        def scale(g2, carry2):
            wvec = plsc.load_gather(wbuf, [lanes + g2 * 16])
            for ee in range(16):
                e = g2 * 16 + ee
                wv = jnp.full((16,), wvec[ee], _f32)
                for kk in range(4):
                    sl = pl.ds(kk * 16, 16)
                    vr[b][e, sl] = vr[b][e, sl] * wv
            return carry2

        lax.fori_loop(0, CHUNK // 16, scale, 0)
        pltpu.async_copy(vr[b], acc_sh.at[idst.at[n]], sa[b], add=True)

    issue(0, 0)

    def pairloop(p, carry):
        n0 = 2 * p
        issue(n0 + 1, 1)
        wait_set(0)
        compute(n0, 0, guard=True)
        issue(n0 + 2, 0)
        wait_set(1)
        compute(n0 + 1, 1, guard=True)
        return carry

    lax.fori_loop(0, (NCHUNKT - 2) // 2, pairloop, 0)
    issue(NCHUNKT - 1, 1)
    wait_set(0)
    compute(NCHUNKT - 2, 0, guard=False)
    wait_set(1)
    compute(NCHUNKT - 1, 1, guard=False)
    wait_acc(0)
    wait_acc(1)
    plsc.subcore_barrier()
    obase = c * NPAD + rbase
    for qq in range(4):
        pltpu.sync_copy(acc_sh.at[pl.ds(rbase + qq * QR, QR), :], stg)
        pltpu.sync_copy(stg, acc_hbm.at[pl.ds(obase + qq * QR, QR), :])


def _sc_agg(l, v, src2d, dst2d, t_all, r_tab, z):
    v_split = jnp.concatenate([v[:, :64], v[:, 64:]], axis=0)
    mesh = plsc.VectorSubcoreMesh(core_axis_name="c", subcore_axis_name="s",
                                  num_cores=NC, num_subcores=NS)
    f = pl.kernel(
        functools.partial(_sc_agg_body, l),
        out_type=jax.ShapeDtypeStruct((NC * NPAD, 64), _f32),
        mesh=mesh,
        compiler_params=pltpu.CompilerParams(needs_layout_passes=False,
                                             use_tc_tiling_on_sc=False),
        scratch_types=[
            pltpu.VMEM((NCHUNKT, CHUNK), _i32), pltpu.VMEM((NCHUNKT, CHUNK), _i32),
            pltpu.VMEM((CHUNK, 64), _f32), pltpu.VMEM((CHUNK, 64), _f32),
            pltpu.VMEM((CHUNK * 4,), _f32), pltpu.VMEM((CHUNK * 4,), _f32),
            pltpu.VMEM((CHUNK, 16), _f32), pltpu.VMEM((CHUNK, 16), _f32),
            pltpu.VMEM((CHUNK,), _f32),
            pltpu.VMEM((ROWS_PER_TILE // 4, 64), _f32),
            pltpu.VMEM((ROWS_PER_TILE // 4, 16), _f32),
            pltpu.VMEM((ROWS_PER_TILE // 4, 16), _f32),
            pltpu.VMEM_SHARED((NPAD, 64), _f32),
            pltpu.VMEM_SHARED((NPAD, 16), _f32),
        ] + [pltpu.SemaphoreType.DMA] * 8)
    acc = f(v_split, src2d, dst2d, t_all, r_tab, z)
    return jnp.concatenate([acc[:NPAD], acc[NPAD:]], axis=1)

def _int0_body(ae_ref, loc_ref, g_ref, v16_ref, wg_ref, bg_ref, wv1_ref,
               ae1_out, v1_out):
    ae = ae_ref[...]
    local = loc_ref[...]
    gate = jax.nn.sigmoid(jnp.sum(ae * wg_ref[...], axis=-1, keepdims=True)
                          + bg_ref[...])
    glob = jnp.dot(g_ref[...], v16_ref[...], preferred_element_type=_f32)
    ae1 = ae + gate * local + (1.0 - gate) * glob
    ae1_out[...] = ae1
    v1_out[...] = jnp.dot(ae1, wv1_ref[...], preferred_element_type=_f32)


def _int1_body(ae_ref, loc_ref, g_ref, v16_ref, wg_ref, bg_ref,
               w1_ref, b1_ref, w2_ref, b2_ref, w3_ref, b3_ref, out_ref):
    ae = ae_ref[...]
    local = loc_ref[...]
    gate = jax.nn.sigmoid(jnp.sum(ae * wg_ref[...], axis=-1, keepdims=True)
                          + bg_ref[...])
    glob = jnp.dot(g_ref[...], v16_ref[...], preferred_element_type=_f32)
    ae2 = ae + gate * local + (1.0 - gate) * glob
    x = jax.nn.relu(jnp.dot(ae2, w1_ref[...], preferred_element_type=_f32) + b1_ref[...])
    x = jax.nn.relu(jnp.dot(x, w2_ref[...], preferred_element_type=_f32) + b2_ref[...])
    out_ref[...] = jnp.dot(x, w3_ref[...], preferred_element_type=_f32) + b3_ref[...]


def _tc_int0(ae_cur, acc, g_tab, v16, lp, wv1):
    full = lambda shp: pl.BlockSpec(shp, lambda i: (0,) * len(shp))
    row = lambda w: pl.BlockSpec((BLK, w), lambda i: (i, 0))
    gspec = pl.BlockSpec((BLK, 16), lambda i: (i, 0))
    ins = [ae_cur, acc, g_tab[:, :16], v16,
           lp['w_gate'].reshape(1, F), lp['b_gate'].reshape(1, 1), wv1]
    in_specs = [row(F), row(F), gspec,
                full((16, F)), full((1, F)), full((1, 1)), full((F, F))]
    return pl.pallas_call(
        _int0_body, grid=(NPAD // BLK,), in_specs=in_specs,
        out_specs=[row(F), row(F)],
        out_shape=[jax.ShapeDtypeStruct((NPAD, F), _f32),
                   jax.ShapeDtypeStruct((NPAD, F), _f32)])(*ins)


def _tc_int1(ae_cur, acc, g_tab, v16, lp, p):
    full = lambda shp: pl.BlockSpec(shp, lambda i: (0,) * len(shp))
    row = lambda w: pl.BlockSpec((BLK, w), lambda i: (i, 0))
    gspec = pl.BlockSpec((BLK, 16), lambda i: (i, 0))
    ins = [ae_cur, acc, g_tab[:, 16:], v16,
           lp['w_gate'].reshape(1, F), lp['b_gate'].reshape(1, 1),
           p['cls1_w'], p['cls1_b'].reshape(1, -1),
           p['cls2_w'], p['cls2_b'].reshape(1, -1),
           p['cls3_w'], p['cls3_b'].reshape(1, -1)]
    in_specs = [row(F), row(F), gspec,
                full((16, F)), full((1, F)), full((1, 1)),
                full((F, 64)), full((1, 64)), full((64, 32)), full((1, 32)),
                full((32, 10)), full((1, 10))]
    return pl.pallas_call(
        _int1_body, grid=(NPAD // BLK,), in_specs=in_specs,
        out_specs=row(10),
        out_shape=jax.ShapeDtypeStruct((NPAD, 10), _f32))(*ins)


def kernel(ae, pe, se, edge_index, K, params):
    del K
    ae = ae.astype(_f32)
    pe = pe.astype(_f32)
    se = se.astype(_f32)
    pe16 = pe[:16]
    se16 = se[:16]
    ae_p = jnp.pad(ae, ((0, NPAD - N), (0, 0)))
    pe_p = jnp.pad(pe, ((0, NPAD - N), (0, 0)))
    se_p = jnp.pad(se, ((0, NPAD - N), (0, 0)))

    src = edge_index[0].astype(_i32)
    dst = edge_index[1].astype(_i32)
    pad_idx = N + (jnp.arange(EPAD - E, dtype=_i32) % (NPAD - N))
    src_p = jnp.concatenate([src, pad_idx]).reshape(NW * NCHUNK, CHUNK)
    dst_p = jnp.concatenate([dst, pad_idx]).reshape(NW * NCHUNK, CHUNK)

    zt = jnp.zeros((ROWS_PER_TILE, 16), _f32)
    zf = jnp.zeros((ROWS_PER_TILE // 4, 64), _f32)

    lay = params['layers']
    ae0, q_tab, k_tab, g_tab, v0 = _tc_prep(ae_p, pe_p, se_p, pe16, se16,
                                            params, lay[0]['Wv_ae'])
    t_all, den = _sc_scores(q_tab, k_tab, src_p, dst_p, zt)

    acc0 = _sc_agg(0, v0, src_p, dst_p, t_all, den, zf)
    ae1, v1 = _tc_int0(ae0, acc0, g_tab, v0[:16], lay[0], lay[1]['Wv_ae'])

    acc1 = _sc_agg(1, v1, src_p, dst_p, t_all, den, zf)
    out = _tc_int1(ae1, acc1, g_tab, v1[:16], lay[1], params)
    return out[:N]
